# baseline (speedup 1.0000x reference)
import functools

import jax
import jax.numpy as jnp
from jax import lax
from jax.experimental import pallas as pl
from jax.experimental.pallas import tpu as pltpu
from jax.experimental.pallas import tpu_sc as plsc

VDIM = 256
HALF = 128
NEXP = 16
BATCH = 1024
TOK = 2
OUT_DIM = 256
NTOK = BATCH * TOK
NC, NS = 2, 16
NW = NC * NS
TOK_PER_W = NTOK // NW


@functools.cache
def _make_gather():
    mesh = plsc.VectorSubcoreMesh(core_axis_name="c", subcore_axis_name="s")

    @functools.partial(
        pl.kernel,
        out_type=jax.ShapeDtypeStruct((NTOK, HALF), jnp.float32),
        mesh=mesh,
        scratch_types=[
            pltpu.VMEM((16,), jnp.int32),
            pltpu.VMEM((TOK_PER_W,), jnp.int32),
            pltpu.VMEM((TOK_PER_W // 2, HALF), jnp.float32),
            pltpu.VMEM((TOK_PER_W // 2, HALF), jnp.float32),
            pltpu.SemaphoreType.DMA,
            pltpu.SemaphoreType.DMA,
        ],
    )
    def gather_rows(cond_hbm, idx_hbm, table_hbm, out_hbm,
                    cond_v, idx_v, rows0_v, rows1_v, gsem, wsem):
        wid = lax.axis_index("s") * NC + lax.axis_index("c")
        base = wid * TOK_PER_W
        half_w = TOK_PER_W // 2
        pltpu.sync_copy(cond_hbm, cond_v)
        off = pl.multiple_of(cond_v[...][0] * HALF, HALF)
        pltpu.sync_copy(idx_hbm.at[pl.ds(base, TOK_PER_W)], idx_v)
        cp0 = pltpu.async_copy(
            table_hbm.at[idx_v.at[pl.ds(0, half_w)], pl.ds(off, HALF)],
            rows0_v, gsem)
        cp1 = pltpu.async_copy(
            table_hbm.at[idx_v.at[pl.ds(half_w, half_w)], pl.ds(off, HALF)],
            rows1_v, gsem)
        cp0.wait()
        w0 = pltpu.async_copy(rows0_v, out_hbm.at[pl.ds(base, half_w)], wsem)
        cp1.wait()
        w1 = pltpu.async_copy(
            rows1_v, out_hbm.at[pl.ds(base + half_w, half_w)], wsem)
        w0.wait()
        w1.wait()

    return gather_rows


def _combine_body(cond_ref, v_ref, m_ref, w_ref, out_ref):
    off = pl.multiple_of(cond_ref[0, 0] * HALF, HALF)
    v0h = v_ref[:BATCH, :]
    v1h = v_ref[BATCH:, :]
    wh = w_ref[:, pl.ds(off, HALF), :].reshape(NEXP * HALF, OUT_DIM)
    blocks = [
        v0h * m_ref[:, e:e + 1] + v1h * m_ref[:, NEXP + e:NEXP + e + 1]
        for e in range(NEXP)
    ]
    a = jnp.concatenate(blocks, axis=1)
    out_ref[...] = jnp.dot(a, wh, preferred_element_type=jnp.float32)


_combine = pl.pallas_call(
    _combine_body,
    out_shape=jax.ShapeDtypeStruct((BATCH, OUT_DIM), jnp.float32),
    in_specs=[
        pl.BlockSpec(memory_space=pltpu.SMEM),
        pl.BlockSpec(),
        pl.BlockSpec(),
        pl.BlockSpec(),
    ],
)


def kernel(indices, scores, W, label, index, weight):
    idx_flat = indices.T.reshape(-1).astype(jnp.int32)
    c = (jnp.asarray(index) == 1).astype(jnp.int32)
    rows = _make_gather()(jnp.broadcast_to(c, (16,)), idx_flat, weight)
    cond = c.reshape(1, 1)
    onehot = (label[:, :, None] == jnp.arange(NEXP, dtype=label.dtype)).astype(
        jnp.float32)
    m = (scores[:, :, None] * onehot).reshape(BATCH, TOK * NEXP)
    return _combine(cond, rows, m, W)


# speedup vs baseline: 1.0463x; 1.0463x over previous
import functools

import jax
import jax.numpy as jnp
from jax import lax
from jax.experimental import pallas as pl
from jax.experimental.pallas import tpu as pltpu
from jax.experimental.pallas import tpu_sc as plsc

VDIM = 256
HALF = 128
NEXP = 16
BATCH = 1024
TOK = 2
OUT_DIM = 256
NTOK = BATCH * TOK
NC, NS = 2, 16
NW = NC * NS
TOK_PER_W = NTOK // NW


@functools.cache
def _make_gather():
    mesh = plsc.VectorSubcoreMesh(core_axis_name="c", subcore_axis_name="s")

    @functools.partial(
        pl.kernel,
        out_type=jax.ShapeDtypeStruct((NTOK, VDIM), jnp.float32),
        mesh=mesh,
        scratch_types=[
            pltpu.VMEM((TOK_PER_W,), jnp.int32),
            pltpu.VMEM((TOK_PER_W // 2, VDIM), jnp.float32),
            pltpu.VMEM((TOK_PER_W // 2, VDIM), jnp.float32),
            pltpu.SemaphoreType.DMA,
            pltpu.SemaphoreType.DMA,
        ],
    )
    def gather_rows(idx_hbm, table_hbm, out_hbm,
                    idx_v, rows0_v, rows1_v, gsem, wsem):
        wid = lax.axis_index("s") * NC + lax.axis_index("c")
        base = wid * TOK_PER_W
        half_w = TOK_PER_W // 2
        pltpu.sync_copy(idx_hbm.at[pl.ds(base, TOK_PER_W)], idx_v)
        cp0 = pltpu.async_copy(
            table_hbm.at[idx_v.at[pl.ds(0, half_w)]], rows0_v, gsem)
        cp1 = pltpu.async_copy(
            table_hbm.at[idx_v.at[pl.ds(half_w, half_w)]], rows1_v, gsem)
        cp0.wait()
        w0 = pltpu.async_copy(rows0_v, out_hbm.at[pl.ds(base, half_w)], wsem)
        cp1.wait()
        w1 = pltpu.async_copy(
            rows1_v, out_hbm.at[pl.ds(base + half_w, half_w)], wsem)
        w0.wait()
        w1.wait()

    return gather_rows


def _combine_body(cond_ref, v_ref, m_ref, w_ref, out_ref):
    off = pl.multiple_of(cond_ref[0, 0] * HALF, HALF)
    v0h = v_ref[:BATCH, pl.ds(off, HALF)]
    v1h = v_ref[BATCH:, pl.ds(off, HALF)]
    wh = w_ref[:, pl.ds(off, HALF), :].reshape(NEXP * HALF, OUT_DIM)
    blocks = [
        v0h * m_ref[:, e:e + 1] + v1h * m_ref[:, NEXP + e:NEXP + e + 1]
        for e in range(NEXP)
    ]
    a = jnp.concatenate(blocks, axis=1)
    out_ref[...] = jnp.dot(a, wh, preferred_element_type=jnp.float32)


_combine = pl.pallas_call(
    _combine_body,
    out_shape=jax.ShapeDtypeStruct((BATCH, OUT_DIM), jnp.float32),
    in_specs=[
        pl.BlockSpec(memory_space=pltpu.SMEM),
        pl.BlockSpec(),
        pl.BlockSpec(),
        pl.BlockSpec(),
    ],
)


def kernel(indices, scores, W, label, index, weight):
    idx_flat = indices.T.reshape(-1).astype(jnp.int32)
    rows = _make_gather()(idx_flat, weight)
    cond = (jnp.asarray(index) == 1).astype(jnp.int32).reshape(1, 1)
    onehot = (label[:, :, None] == jnp.arange(NEXP, dtype=label.dtype)).astype(
        jnp.float32)
    m = (scores[:, :, None] * onehot).reshape(BATCH, TOK * NEXP)
    return _combine(cond, rows, m, W)
